# Initial kernel scaffold; baseline (speedup 1.0000x reference)
import jax
import jax.numpy as jnp
from jax.experimental import pallas as pl


def kernel(x, edge_index, edge_weight, params):
    raise NotImplementedError("write your pallas kernel here")


# speedup vs baseline: 31.8732x; 31.8732x over previous
import functools

import jax
import jax.numpy as jnp
from jax import lax
from jax.experimental import pallas as pl
from jax.experimental.pallas import tpu as pltpu, tpu_sc as plsc

NN = 10000
EE = 320000
D = 128
NHEADS = 16
NCH = 8
NEGS = 0.2
NC, NS, LANES = 2, 16, 16
NW = NC * NS
NPAD = 10240
RNG = NPAD // NW
ECHUNK = 128
EPAD = EE + 256
ROWB = 1024
EPS_DEN = 1e-16
EPS_LN = 1e-5


def _cm_vec(v):
    return v.reshape(NHEADS, NCH).T.reshape(D)


def _cm_cols(w):
    return w.reshape(-1, NHEADS, NCH).transpose(0, 2, 1).reshape(w.shape[0], D)


def _cm_rows(w):
    return _cm_cols(w.T).T


def _gelu(y):
    return 0.5 * y * (1.0 + lax.erf(y * (2.0 ** -0.5)))


def _ln_rows(h, g, b):
    m = jnp.mean(h, axis=1, keepdims=True)
    v = jnp.mean((h - m) ** 2, axis=1, keepdims=True)
    return (h - m) / jnp.sqrt(v + EPS_LN) * g + b


def _t0_body(x_ref, w_ref, b_ref, g_ref, bb_ref, o_ref):
    y = jnp.dot(x_ref[...], w_ref[...], preferred_element_type=jnp.float32)
    y = y + b_ref[...]
    o_ref[...] = _ln_rows(_gelu(y), g_ref[...], bb_ref[...])


def _ta_body(h_ref, g_ref, b_ref, wl_ref, bl_ref, wr_ref, br_ref,
             xl_ref, xr_ref):
    hn = _ln_rows(h_ref[...], g_ref[...], b_ref[...])
    xl_ref[...] = jnp.dot(hn, wl_ref[...],
                          preferred_element_type=jnp.float32) + bl_ref[...]
    xr_ref[...] = jnp.dot(hn, wr_ref[...],
                          preferred_element_type=jnp.float32) + br_ref[...]


def _tc_post_body(h_ref, acc_ref, den_ref, bias_ref, o_ref):
    den8 = jnp.concatenate([den_ref[...]] * NCH, axis=1)
    m = acc_ref[...] / (den8 + EPS_DEN) + bias_ref[...]
    o_ref[...] = h_ref[...] + _gelu(m)


def _tf_body(h_ref, gw_ref, gb_ref, g_ref, b_ref, o_ref):
    h = h_ref[...]
    rid = lax.broadcasted_iota(jnp.int32, (NPAD, 1), 0)
    hm = jnp.where(rid < NN, h, 0.0)
    ctx = jnp.sum(hm, axis=0, keepdims=True) * (1.0 / NN)
    gate = jax.nn.sigmoid(
        jnp.dot(ctx, gw_ref[...], preferred_element_type=jnp.float32)
        + gb_ref[...])
    hh = h + gate * ctx
    o_ref[...] = _ln_rows(hh, g_ref[...], b_ref[...])


_GRID = NPAD // ROWB
_row_spec = pl.BlockSpec((ROWB, D), lambda i: (i, 0))
_den_spec = pl.BlockSpec((ROWB, NHEADS), lambda i: (i, 0))
_w_spec = pl.BlockSpec((D, D), lambda i: (0, 0))
_v_spec = pl.BlockSpec((1, D), lambda i: (0, 0))
_f32 = jnp.float32

_t0_call = pl.pallas_call(
    _t0_body,
    grid=(_GRID,),
    in_specs=[_row_spec, _w_spec, _v_spec, _v_spec, _v_spec],
    out_specs=_row_spec,
    out_shape=jax.ShapeDtypeStruct((NPAD, D), _f32),
)

_ta_call = pl.pallas_call(
    _ta_body,
    grid=(_GRID,),
    in_specs=[_row_spec, _v_spec, _v_spec, _w_spec, _v_spec, _w_spec, _v_spec],
    out_specs=[_row_spec, _row_spec],
    out_shape=[jax.ShapeDtypeStruct((NPAD, D), _f32),
               jax.ShapeDtypeStruct((NPAD, D), _f32)],
)

_tc_post_call = pl.pallas_call(
    _tc_post_body,
    grid=(_GRID,),
    in_specs=[_row_spec, _row_spec, _den_spec, _v_spec],
    out_specs=_row_spec,
    out_shape=jax.ShapeDtypeStruct((NPAD, D), _f32),
)

_tf_call = pl.pallas_call(
    _tf_body,
    in_specs=[pl.BlockSpec((NPAD, D), lambda: (0, 0)),
              pl.BlockSpec((D, D), lambda: (0, 0)),
              pl.BlockSpec((1, D), lambda: (0, 0)),
              pl.BlockSpec((1, D), lambda: (0, 0)),
              pl.BlockSpec((1, D), lambda: (0, 0))],
    out_specs=pl.BlockSpec((NPAD, D), lambda: (0, 0)),
    out_shape=jax.ShapeDtypeStruct((NPAD, D), _f32),
)


def _sc_edge_body(xl_hbm, xr_hbm, src_hbm, dst_hbm, ew_hbm, bnd_hbm,
                  erow_hbm, att_hbm, acc_hbm, den_hbm,
                  xr_v, acc_v, m_v, den_v, xlbuf, idx_v, dstbuf, ewbuf,
                  bnd_v, erow_v, att_v, sem):
    cid = lax.axis_index("c")
    sid = lax.axis_index("s")
    wid = sid * NC + cid
    ns = wid * RNG

    pltpu.sync_copy(bnd_hbm, bnd_v)
    pltpu.sync_copy(erow_hbm, erow_v)
    pltpu.sync_copy(att_hbm, att_v)
    pltpu.sync_copy(xr_hbm.at[pl.ds(ns * D, RNG * D)], xr_v)

    iot = lax.iota(jnp.int32, 16)
    zero = jnp.zeros((16,), _f32)
    mfill = jnp.full((16,), -1e30, _f32)

    def _init_nd(i, _):
        m_v[pl.ds(i * 16, 16)] = mfill
        den_v[pl.ds(i * 16, 16)] = zero
        return 0
    lax.fori_loop(0, RNG, _init_nd, 0)

    def _init_acc(i, _):
        acc_v[pl.ds(i * 16, 16)] = zero
        return 0
    lax.fori_loop(0, RNG * NCH, _init_acc, 0)

    ecs = [plsc.load_gather(erow_v, [c * 16 + iot]) for c in range(NCH)]
    acs = [plsc.load_gather(att_v, [c * 16 + iot]) for c in range(NCH)]

    def _getb(i):
        v = plsc.load_gather(bnd_v, [jnp.full((16,), i, jnp.int32)])
        return jnp.max(v)

    lo = _getb(wid)
    hi = _getb(wid + 1)
    base8 = lo - lax.rem(lo, 8)
    nch = (hi - base8 + ECHUNK - 1) // ECHUNK

    def _chunk(g, _):
        cb = pl.multiple_of(base8 + g * ECHUNK, 8)
        pltpu.sync_copy(src_hbm.at[pl.ds(cb, ECHUNK)], idx_v)
        pltpu.sync_copy(dst_hbm.at[pl.ds(cb, ECHUNK)], dstbuf)
        pltpu.sync_copy(ew_hbm.at[pl.ds(cb, ECHUNK)], ewbuf)
        pltpu.async_copy(xl_hbm.at[idx_v], xlbuf, sem).wait()
        elo = jnp.maximum(lo, cb) - cb
        ehi = jnp.minimum(hi, cb + ECHUNK) - cb

        def _edge(e, _):
            sp = jnp.full((16,), e, jnp.int32)
            dlv = plsc.load_gather(dstbuf, [sp]) - ns
            ewv = plsc.load_gather(ewbuf, [sp])
            d16 = dlv * 16 + iot
            d128 = dlv * D + iot
            logit = zero
            xls = []
            for c in range(NCH):
                xl_c = xlbuf[e, pl.ds(c * 16, 16)]
                xr_c = plsc.load_gather(xr_v, [d128 + c * 16])
                t = xl_c + xr_c + ewv * ecs[c]
                z = jnp.maximum(t, NEGS * t)
                logit = logit + z * acs[c]
                xls.append(xl_c)
            m_old = plsc.load_gather(m_v, [d16])
            m_new = jnp.maximum(m_old, logit)
            r = jnp.exp(m_old - m_new)
            ex = jnp.exp(logit - m_new)
            plsc.store_scatter(m_v, [d16], m_new)
            d_old = plsc.load_gather(den_v, [d16])
            plsc.store_scatter(den_v, [d16], d_old * r + ex)
            for c in range(NCH):
                a_old = plsc.load_gather(acc_v, [d128 + c * 16])
                plsc.store_scatter(acc_v, [d128 + c * 16],
                                   a_old * r + xls[c] * ex)
            return 0

        lax.fori_loop(elo, ehi, _edge, 0)
        return 0

    lax.fori_loop(0, nch, _chunk, 0)

    pltpu.sync_copy(acc_v, acc_hbm.at[pl.ds(ns * D, RNG * D)])
    pltpu.sync_copy(den_v, den_hbm.at[pl.ds(ns * NHEADS, RNG * NHEADS)])


_sc_edge_call = pl.kernel(
    _sc_edge_body,
    out_type=[jax.ShapeDtypeStruct((NPAD * D,), _f32),
              jax.ShapeDtypeStruct((NPAD * NHEADS,), _f32)],
    mesh=plsc.VectorSubcoreMesh(core_axis_name="c", subcore_axis_name="s"),
    compiler_params=pltpu.CompilerParams(needs_layout_passes=False),
    scratch_types=[
        pltpu.VMEM((RNG * D,), _f32),
        pltpu.VMEM((RNG * D,), _f32),
        pltpu.VMEM((RNG * NHEADS,), _f32),
        pltpu.VMEM((RNG * NHEADS,), _f32),
        pltpu.VMEM((ECHUNK, D), _f32),
        pltpu.VMEM((ECHUNK,), jnp.int32),
        pltpu.VMEM((ECHUNK,), jnp.int32),
        pltpu.VMEM((ECHUNK,), _f32),
        pltpu.VMEM((40,), jnp.int32),
        pltpu.VMEM((D,), _f32),
        pltpu.VMEM((D,), _f32),
        pltpu.SemaphoreType.DMA,
    ],
)


def kernel(x, edge_index, edge_weight, params):
    src = edge_index[0]
    dst = edge_index[1]
    order = jnp.argsort(dst)
    src_s = jnp.take(src, order)
    dst_s = jnp.take(dst, order)
    ew_s = jnp.take(edge_weight[:, 0], order)
    npadE = EPAD - EE
    src_p = jnp.concatenate([src_s, jnp.zeros((npadE,), jnp.int32)])
    dst_p = jnp.concatenate([dst_s,
                             jnp.full((npadE,), NPAD - 1, jnp.int32)])
    ew_p = jnp.concatenate([ew_s, jnp.zeros((npadE,), _f32)])
    bnd = jnp.searchsorted(dst_s, jnp.arange(0, NPAD + 1, RNG)).astype(
        jnp.int32)
    bnd = jnp.concatenate([bnd, jnp.zeros((40 - NW - 1,), jnp.int32)])

    p = params
    in_w = _cm_cols(p["in_W"])
    in_b = _cm_vec(p["in_b"]).reshape(1, D)
    in_g = _cm_vec(p["in_ln_g"]).reshape(1, D)
    in_bb = _cm_vec(p["in_ln_b"]).reshape(1, D)
    x_p = jnp.pad(x, ((0, NPAD - NN), (0, 0)))

    h = _t0_call(x_p, in_w, in_b, in_g, in_bb)

    for lp in p["layers"]:
        g = _cm_vec(lp["ln_g"]).reshape(1, D)
        b = _cm_vec(lp["ln_b"]).reshape(1, D)
        wl = _cm_rows(_cm_cols(lp["lin_l_W"]))
        bl = _cm_vec(lp["lin_l_b"]).reshape(1, D)
        wr = _cm_rows(_cm_cols(lp["lin_r_W"]))
        br = _cm_vec(lp["lin_r_b"]).reshape(1, D)
        erow = _cm_vec(lp["lin_e_W"][0])
        att = lp["att"].T.reshape(D)
        bias = _cm_vec(lp["bias"]).reshape(1, D)

        xl, xr = _ta_call(h, g, b, wl, bl, wr, br)
        acc, den = _sc_edge_call(xl, xr.reshape(NPAD * D), src_p, dst_p,
                                 ew_p, bnd, erow, att)
        h = _tc_post_call(h, acc.reshape(NPAD, D),
                          den.reshape(NPAD, NHEADS), bias)

    gw = _cm_rows(_cm_cols(p["gate_W"]))
    gb = _cm_vec(p["gate_b"]).reshape(1, D)
    fg = _cm_vec(p["fin_ln_g"]).reshape(1, D)
    fb = _cm_vec(p["fin_ln_b"]).reshape(1, D)
    h = _tf_call(h, gw, gb, fg, fb)

    out = h[:NN].reshape(NN, NCH, NHEADS).transpose(0, 2, 1).reshape(NN, D)
    return out
